# baseline (speedup 1.0000x reference)
import functools

import jax
import jax.numpy as jnp
from jax import lax
from jax.experimental import pallas as pl
from jax.experimental.pallas import tpu as pltpu
from jax.experimental.pallas import tpu_sc as plsc

D = 768
H = 1536
E = 8
T = 2048
K = 2
B = 512
NB = 16
NPAD = NB * B
NBE = 64
CH = 512
NCH = K * T // CH
NC, NS = 2, 16
NW = NC * NS
TPW = T // NW


def _gate_body(x_ref, gw_ref, gb_ref,
               pos0_ref, pos1_ref, w0_ref, w1_ref, be_ref, na_ref):
    x = x_ref[...]
    logits = jnp.dot(x, gw_ref[...], preferred_element_type=jnp.float32)
    logits = logits + gb_ref[...]
    m = jnp.max(logits, axis=1, keepdims=True)
    ex = jnp.exp(logits - m)
    probs = ex / jnp.sum(ex, axis=1, keepdims=True)

    iota_e = lax.broadcasted_iota(jnp.int32, (T, E), 1)
    p0 = jnp.max(probs, axis=1, keepdims=True)
    e0 = jnp.min(jnp.where(probs == p0, iota_e, E), axis=1, keepdims=True)
    probs1 = jnp.where(iota_e == e0, -1.0, probs)
    p1 = jnp.max(probs1, axis=1, keepdims=True)
    e1 = jnp.min(jnp.where(probs1 == p1, iota_e, E), axis=1, keepdims=True)
    s = p0 + p1
    w0_ref[...] = jnp.broadcast_to(p0 / s, (T, 128))
    w1_ref[...] = jnp.broadcast_to(p1 / s, (T, 128))

    r_i = lax.broadcasted_iota(jnp.int32, (CH, CH), 0)
    c_i = lax.broadcasted_iota(jnp.int32, (CH, CH), 1)
    lt = (r_i > c_i).astype(jnp.float32)
    iota_ce = lax.broadcasted_iota(jnp.int32, (CH, E), 1)

    offs = jnp.zeros((1, E), jnp.float32)
    ohs, ranks = [], []
    for part in (e0, e1):
        for c in range(T // CH):
            ec = lax.slice(part, (c * CH, 0), ((c + 1) * CH, 1))
            oh = (ec == iota_ce).astype(jnp.float32)
            rk = jnp.dot(lt, oh, preferred_element_type=jnp.float32) + offs
            ohs.append(oh)
            ranks.append(jnp.sum(oh * rk, axis=1, keepdims=True))
            offs = offs + jnp.sum(oh, axis=0, keepdims=True)

    counts_i = offs.astype(jnp.int32)
    pad_i = ((counts_i + (B - 1)) // B) * B
    pad_f = pad_i.astype(jnp.float32)
    r8 = lax.broadcasted_iota(jnp.int32, (E, E), 0)
    c8 = lax.broadcasted_iota(jnp.int32, (E, E), 1)
    su = (r8 < c8).astype(jnp.float32)
    offs_pad = jnp.dot(pad_f, su, preferred_element_type=jnp.float32)

    pos = [ranks[i] + jnp.sum(ohs[i] * offs_pad, axis=1, keepdims=True)
           for i in range(NCH)]
    n = NCH // 2
    pos0_ref[...] = jnp.concatenate(pos[:n], axis=0).astype(jnp.int32)
    pos1_ref[...] = jnp.concatenate(pos[n:], axis=0).astype(jnp.int32)

    ends = offs_pad + pad_f
    bstart = (lax.broadcasted_iota(jnp.int32, (NBE, 1), 0) * B)
    done = (ends <= bstart.astype(jnp.float32)).astype(jnp.int32)
    be_ref[...] = jnp.minimum(jnp.sum(done, axis=1, keepdims=True), E - 1)
    na_ref[...] = jnp.full((1, 1), 0, jnp.int32) + (jnp.sum(pad_i) // B)


_gate_call = pl.pallas_call(
    _gate_body,
    out_shape=(
        jax.ShapeDtypeStruct((T, 1), jnp.int32),
        jax.ShapeDtypeStruct((T, 1), jnp.int32),
        jax.ShapeDtypeStruct((T, 128), jnp.float32),
        jax.ShapeDtypeStruct((T, 128), jnp.float32),
        jax.ShapeDtypeStruct((NBE, 1), jnp.int32),
        jax.ShapeDtypeStruct((1, 1), jnp.int32),
    ),
)


def _dispatch_body(x_hbm, pos0_hbm, pos1_hbm, w0_hbm, w1_hbm,
                   xd_hbm, ws_hbm, xb, i0, i1, wb0, wb1, sem):
    wid = lax.axis_index("s") * NC + lax.axis_index("c")
    base = wid * TPW
    pltpu.sync_copy(x_hbm.at[pl.ds(base, TPW)], xb)
    pltpu.sync_copy(pos0_hbm.at[pl.ds(base, TPW)], i0)
    pltpu.sync_copy(pos1_hbm.at[pl.ds(base, TPW)], i1)
    pltpu.sync_copy(w0_hbm.at[pl.ds(base, TPW)], wb0)
    pltpu.sync_copy(w1_hbm.at[pl.ds(base, TPW)], wb1)
    c0 = pltpu.async_copy(xb, xd_hbm.at[i0], sem)
    c1 = pltpu.async_copy(xb, xd_hbm.at[i1], sem)
    c2 = pltpu.async_copy(wb0, ws_hbm.at[i0], sem)
    c3 = pltpu.async_copy(wb1, ws_hbm.at[i1], sem)
    c0.wait(); c1.wait(); c2.wait(); c3.wait()


def _ffn_body(be_sref, na_sref, xd_ref, w1_ref, b1_ref, w2_ref, b2_ref,
              ws_ref, yd_ref):
    i = pl.program_id(0)

    @pl.when(i < na_sref[0])
    def _():
        xb = xd_ref[...].astype(jnp.bfloat16)
        w1b = w1_ref[0].astype(jnp.bfloat16)
        h = jnp.dot(xb, w1b, preferred_element_type=jnp.float32)
        h = jnp.maximum(h + b1_ref[0], 0.0)
        w2b = w2_ref[0].astype(jnp.bfloat16)
        y = jnp.dot(h.astype(jnp.bfloat16), w2b,
                    preferred_element_type=jnp.float32)
        yd_ref[...] = (y + b2_ref[0]) * ws_ref[..., 0:1]


_ffn_call = pl.pallas_call(
    _ffn_body,
    grid_spec=pltpu.PrefetchScalarGridSpec(
        num_scalar_prefetch=2,
        grid=(NB,),
        in_specs=[
            pl.BlockSpec((B, D), lambda i, be, na: (i, 0)),
            pl.BlockSpec((1, D, H), lambda i, be, na: (be[i], 0, 0)),
            pl.BlockSpec((1, 1, H), lambda i, be, na: (be[i], 0, 0)),
            pl.BlockSpec((1, H, D), lambda i, be, na: (be[i], 0, 0)),
            pl.BlockSpec((1, 1, D), lambda i, be, na: (be[i], 0, 0)),
            pl.BlockSpec((B, 128), lambda i, be, na: (i, 0)),
        ],
        out_specs=pl.BlockSpec((B, D), lambda i, be, na: (i, 0)),
    ),
    out_shape=jax.ShapeDtypeStruct((NPAD, D), jnp.float32),
)


def _combine_body(yd_hbm, pos0_hbm, pos1_hbm, out_hbm, z0, z1, i0, i1, sem):
    wid = lax.axis_index("s") * NC + lax.axis_index("c")
    base = wid * TPW
    pltpu.sync_copy(pos0_hbm.at[pl.ds(base, TPW)], i0)
    pltpu.sync_copy(pos1_hbm.at[pl.ds(base, TPW)], i1)
    g0 = pltpu.async_copy(yd_hbm.at[i0], z0, sem)
    g1 = pltpu.async_copy(yd_hbm.at[i1], z1, sem)
    g0.wait()
    g1.wait()

    def row(r, carry):
        for c in range(D // 16):
            sl = pl.ds(c * 16, 16)
            z0[r, sl] = z0[r, sl] + z1[r, sl]
        return carry

    lax.fori_loop(0, TPW, row, 0)
    pltpu.sync_copy(z0, out_hbm.at[pl.ds(base, TPW)])


@functools.cache
def _sc_kernels():
    mesh = plsc.VectorSubcoreMesh(
        core_axis_name="c", subcore_axis_name="s",
        num_cores=NC, num_subcores=NS)
    dispatch = pl.kernel(
        _dispatch_body,
        out_type=(
            jax.ShapeDtypeStruct((NPAD, D), jnp.float32),
            jax.ShapeDtypeStruct((NPAD, 128), jnp.float32),
        ),
        mesh=mesh,
        scratch_types=[
            pltpu.VMEM((TPW, D), jnp.float32),
            pltpu.VMEM((TPW,), jnp.int32),
            pltpu.VMEM((TPW,), jnp.int32),
            pltpu.VMEM((TPW, 128), jnp.float32),
            pltpu.VMEM((TPW, 128), jnp.float32),
            pltpu.SemaphoreType.DMA,
        ],
    )
    combine = pl.kernel(
        _combine_body,
        out_type=jax.ShapeDtypeStruct((T, D), jnp.float32),
        mesh=mesh,
        scratch_types=[
            pltpu.VMEM((TPW, D), jnp.float32),
            pltpu.VMEM((TPW, D), jnp.float32),
            pltpu.VMEM((TPW,), jnp.int32),
            pltpu.VMEM((TPW,), jnp.int32),
            pltpu.SemaphoreType.DMA,
        ],
    )
    return dispatch, combine


def kernel(x, gate_W, gate_b, W1, b1, W2, b2):
    dispatch, combine = _sc_kernels()
    x2d = x.reshape(T, D)
    pos0, pos1, w0, w1, be, na = _gate_call(x2d, gate_W, gate_b.reshape(1, E))
    pos0 = pos0.reshape(T)
    pos1 = pos1.reshape(T)
    xd, ws = dispatch(x2d, pos0, pos1, w0, w1)
    yd = _ffn_call(be.reshape(NBE), na.reshape(1), xd,
                   W1, b1.reshape(E, 1, H), W2, b2.reshape(E, 1, D), ws)
    out = combine(yd, pos0, pos1)
    return out.reshape(1, T, D)


# speedup vs baseline: 1.0091x; 1.0091x over previous
import functools

import jax
import jax.numpy as jnp
from jax import lax
from jax.experimental import pallas as pl
from jax.experimental.pallas import tpu as pltpu
from jax.experimental.pallas import tpu_sc as plsc

D = 768
H = 1536
E = 8
T = 2048
K = 2
B = 512
NB = 16
NPAD = NB * B
NBE = 64
CH = 512
NCH = K * T // CH
NC, NS = 2, 16
NW = NC * NS
TPW = T // NW


def _gate_body(x_ref, gw_ref, gb_ref,
               pos0_ref, pos1_ref, w0_ref, w1_ref, be_ref, na_ref):
    x = x_ref[...]
    logits = jnp.dot(x, gw_ref[...], preferred_element_type=jnp.float32)
    logits = logits + gb_ref[...]
    m = jnp.max(logits, axis=1, keepdims=True)
    ex = jnp.exp(logits - m)
    probs = ex / jnp.sum(ex, axis=1, keepdims=True)

    iota_e = lax.broadcasted_iota(jnp.int32, (T, E), 1)
    p0 = jnp.max(probs, axis=1, keepdims=True)
    e0 = jnp.min(jnp.where(probs == p0, iota_e, E), axis=1, keepdims=True)
    probs1 = jnp.where(iota_e == e0, -1.0, probs)
    p1 = jnp.max(probs1, axis=1, keepdims=True)
    e1 = jnp.min(jnp.where(probs1 == p1, iota_e, E), axis=1, keepdims=True)
    s = p0 + p1
    w0_ref[...] = jnp.broadcast_to(p0 / s, (T, 16))
    w1_ref[...] = jnp.broadcast_to(p1 / s, (T, 16))

    r_i = lax.broadcasted_iota(jnp.int32, (CH, CH), 0)
    c_i = lax.broadcasted_iota(jnp.int32, (CH, CH), 1)
    lt = (r_i > c_i).astype(jnp.float32)
    iota_ce = lax.broadcasted_iota(jnp.int32, (CH, E), 1)

    offs = jnp.zeros((1, E), jnp.float32)
    ohs, ranks = [], []
    for part in (e0, e1):
        for c in range(T // CH):
            ec = lax.slice(part, (c * CH, 0), ((c + 1) * CH, 1))
            oh = (ec == iota_ce).astype(jnp.float32)
            rk = jnp.dot(lt, oh, preferred_element_type=jnp.float32) + offs
            ohs.append(oh)
            ranks.append(jnp.sum(oh * rk, axis=1, keepdims=True))
            offs = offs + jnp.sum(oh, axis=0, keepdims=True)

    counts_i = offs.astype(jnp.int32)
    pad_i = ((counts_i + (B - 1)) // B) * B
    pad_f = pad_i.astype(jnp.float32)
    r8 = lax.broadcasted_iota(jnp.int32, (E, E), 0)
    c8 = lax.broadcasted_iota(jnp.int32, (E, E), 1)
    su = (r8 < c8).astype(jnp.float32)
    offs_pad = jnp.dot(pad_f, su, preferred_element_type=jnp.float32)

    pos = [ranks[i] + jnp.sum(ohs[i] * offs_pad, axis=1, keepdims=True)
           for i in range(NCH)]
    n = NCH // 2
    pos0_ref[...] = jnp.concatenate(pos[:n], axis=0).astype(jnp.int32)
    pos1_ref[...] = jnp.concatenate(pos[n:], axis=0).astype(jnp.int32)

    ends = offs_pad + pad_f
    bstart = (lax.broadcasted_iota(jnp.int32, (NBE, 1), 0) * B)
    done = (ends <= bstart.astype(jnp.float32)).astype(jnp.int32)
    be_ref[...] = jnp.minimum(jnp.sum(done, axis=1, keepdims=True), E - 1)
    na_ref[...] = jnp.full((1, 1), 0, jnp.int32) + (jnp.sum(pad_i) // B)


_gate_call = pl.pallas_call(
    _gate_body,
    out_shape=(
        jax.ShapeDtypeStruct((T, 1), jnp.int32),
        jax.ShapeDtypeStruct((T, 1), jnp.int32),
        jax.ShapeDtypeStruct((T, 16), jnp.float32),
        jax.ShapeDtypeStruct((T, 16), jnp.float32),
        jax.ShapeDtypeStruct((NBE, 1), jnp.int32),
        jax.ShapeDtypeStruct((1, 1), jnp.int32),
    ),
)


def _dispatch_body(x_hbm, pos0_hbm, pos1_hbm, xd_hbm, xb, i0, i1, sem):
    wid = lax.axis_index("s") * NC + lax.axis_index("c")
    base = wid * TPW
    pltpu.sync_copy(x_hbm.at[pl.ds(base, TPW)], xb)
    pltpu.sync_copy(pos0_hbm.at[pl.ds(base, TPW)], i0)
    pltpu.sync_copy(pos1_hbm.at[pl.ds(base, TPW)], i1)
    c0 = pltpu.async_copy(xb, xd_hbm.at[i0], sem)
    c1 = pltpu.async_copy(xb, xd_hbm.at[i1], sem)
    c0.wait(); c1.wait()


def _ffn_body(be_sref, na_sref, xd_ref, w1_ref, b1_ref, w2_ref, b2_ref,
              yd_ref):
    i = pl.program_id(0)

    @pl.when(i < na_sref[0])
    def _():
        xb = xd_ref[...].astype(jnp.bfloat16)
        w1b = w1_ref[0].astype(jnp.bfloat16)
        h = jnp.dot(xb, w1b, preferred_element_type=jnp.float32)
        h = jnp.maximum(h + b1_ref[0], 0.0)
        w2b = w2_ref[0].astype(jnp.bfloat16)
        y = jnp.dot(h.astype(jnp.bfloat16), w2b,
                    preferred_element_type=jnp.float32)
        yd_ref[...] = y + b2_ref[0]


_ffn_call = pl.pallas_call(
    _ffn_body,
    grid_spec=pltpu.PrefetchScalarGridSpec(
        num_scalar_prefetch=2,
        grid=(NB,),
        in_specs=[
            pl.BlockSpec((B, D), lambda i, be, na: (i, 0)),
            pl.BlockSpec((1, D, H), lambda i, be, na: (be[i], 0, 0)),
            pl.BlockSpec((1, 1, H), lambda i, be, na: (be[i], 0, 0)),
            pl.BlockSpec((1, H, D), lambda i, be, na: (be[i], 0, 0)),
            pl.BlockSpec((1, 1, D), lambda i, be, na: (be[i], 0, 0)),
        ],
        out_specs=pl.BlockSpec((B, D), lambda i, be, na: (i, 0)),
    ),
    out_shape=jax.ShapeDtypeStruct((NPAD, D), jnp.float32),
)


def _combine_body(yd_hbm, pos0_hbm, pos1_hbm, w0_hbm, w1_hbm, out_hbm,
                  z0, z1, i0, i1, wb0, wb1, sem):
    wid = lax.axis_index("s") * NC + lax.axis_index("c")
    base = wid * TPW
    pltpu.sync_copy(pos0_hbm.at[pl.ds(base, TPW)], i0)
    pltpu.sync_copy(pos1_hbm.at[pl.ds(base, TPW)], i1)
    pltpu.sync_copy(w0_hbm.at[pl.ds(base, TPW)], wb0)
    pltpu.sync_copy(w1_hbm.at[pl.ds(base, TPW)], wb1)
    g0 = pltpu.async_copy(yd_hbm.at[i0], z0, sem)
    g1 = pltpu.async_copy(yd_hbm.at[i1], z1, sem)
    g0.wait()
    g1.wait()

    def row(r, carry):
        w0v = wb0[r, :]
        w1v = wb1[r, :]
        for c in range(D // 16):
            sl = pl.ds(c * 16, 16)
            z0[r, sl] = z0[r, sl] * w0v + z1[r, sl] * w1v
        return carry

    lax.fori_loop(0, TPW, row, 0)
    pltpu.sync_copy(z0, out_hbm.at[pl.ds(base, TPW)])


@functools.cache
def _sc_kernels():
    mesh = plsc.VectorSubcoreMesh(
        core_axis_name="c", subcore_axis_name="s",
        num_cores=NC, num_subcores=NS)
    dispatch = pl.kernel(
        _dispatch_body,
        out_type=jax.ShapeDtypeStruct((NPAD, D), jnp.float32),
        mesh=mesh,
        scratch_types=[
            pltpu.VMEM((TPW, D), jnp.float32),
            pltpu.VMEM((TPW,), jnp.int32),
            pltpu.VMEM((TPW,), jnp.int32),
            pltpu.SemaphoreType.DMA,
        ],
    )
    combine = pl.kernel(
        _combine_body,
        out_type=jax.ShapeDtypeStruct((T, D), jnp.float32),
        mesh=mesh,
        scratch_types=[
            pltpu.VMEM((TPW, D), jnp.float32),
            pltpu.VMEM((TPW, D), jnp.float32),
            pltpu.VMEM((TPW,), jnp.int32),
            pltpu.VMEM((TPW,), jnp.int32),
            pltpu.VMEM((TPW, 16), jnp.float32),
            pltpu.VMEM((TPW, 16), jnp.float32),
            pltpu.SemaphoreType.DMA,
        ],
    )
    return dispatch, combine


def kernel(x, gate_W, gate_b, W1, b1, W2, b2):
    dispatch, combine = _sc_kernels()
    x2d = x.reshape(T, D)
    pos0, pos1, w0, w1, be, na = _gate_call(x2d, gate_W, gate_b.reshape(1, E))
    pos0 = pos0.reshape(T)
    pos1 = pos1.reshape(T)
    xd = dispatch(x2d, pos0, pos1)
    yd = _ffn_call(be.reshape(NBE), na.reshape(1), xd,
                   W1, b1.reshape(E, 1, H), W2, b2.reshape(E, 1, D))
    out = combine(yd, pos0, pos1, w0, w1)
    return out.reshape(1, T, D)
